# baseline (speedup 1.0000x reference)
import functools

import jax
import jax.numpy as jnp
from jax import lax
from jax.experimental import pallas as pl
from jax.experimental.pallas import tpu as pltpu
from jax.experimental.pallas import tpu_sc as plsc

NUM_EMB = 1_000_000
DIM = 64
BATCH = 16384
HIST = 20
B_TOTAL = BATCH * HIST
NW = 32
CHUNK = 128
NCHUNK = HIST * (BATCH // NW // CHUNK)
BC_PER_W = BATCH // NW // CHUNK
NBUF = 4
NBUF_R = 6
NBUF_G = 5

NBLK = NUM_EMB // CHUNK + 1
NBLK_W = NBLK // NW
NBLK_REM = NBLK - NW * NBLK_W
TBL_ROWS = NBLK * CHUNK

_iota16 = lambda: lax.iota(jnp.int32, 16)


def _make_transpose():
    mesh = plsc.VectorSubcoreMesh(core_axis_name="c", subcore_axis_name="s")

    @functools.partial(
        pl.kernel,
        mesh=mesh,
        out_type=jax.ShapeDtypeStruct((TBL_ROWS, CHUNK), jnp.float32),
        scratch_types=[
            pltpu.VMEM((NBUF_R, DIM, CHUNK), jnp.float32),
            pltpu.VMEM((NBUF, CHUNK, CHUNK), jnp.float32),
            pltpu.SemaphoreType.DMA((NBUF_R,)),
            pltpu.SemaphoreType.DMA((NBUF,)),
        ],
        compiler_params=pltpu.CompilerParams(
            use_tc_tiling_on_sc=True,
            needs_layout_passes=False,
            disable_bounds_checks=True,
        ),
    )
    def transpose_kernel(wt_hbm, out_hbm, col_v, row_v, gsem, ssem):
        wid = lax.axis_index("s") * 2 + lax.axis_index("c")
        niter = NBLK_W + jnp.where(wid < NBLK_REM, 1, 0)

        def col0_of(j):
            return (j * NW + wid) * CHUNK

        for b in range(NBUF_R):
            pltpu.async_copy(
                wt_hbm.at[:, pl.ds(col0_of(b), CHUNK)], col_v.at[b],
                gsem.at[b],
            )

        iota = _iota16()
        zero = iota * 0
        igs = [iota + ig * 16 for ig in range(8)]

        def body(j, carry):
            grb = lax.rem(j, NBUF_R)
            rb = lax.rem(j, NBUF)
            col0 = col0_of(j)
            pltpu.make_async_copy(
                wt_hbm.at[:, pl.ds(0, CHUNK)], col_v.at[grb], gsem.at[grb]
            ).wait()
            @pl.when(j >= NBUF)
            def _():
                pltpu.make_async_copy(
                    row_v.at[rb], out_hbm.at[pl.ds(0, CHUNK)], ssem.at[rb]
                ).wait()
            src = col_v.at[grb]
            dst = row_v.at[rb]

            @plsc.parallel_loop(0, DIM, unroll=8)
            def _transpose_block(d):
                dvec = zero + d
                for ig in range(8):
                    v = src[d, pl.ds(ig * 16, 16)]
                    plsc.store_scatter(dst, [igs[ig], dvec], v)
            pltpu.async_copy(
                row_v.at[rb], out_hbm.at[pl.ds(col0, CHUNK)], ssem.at[rb]
            )
            @pl.when(j + NBUF_R < niter)
            def _():
                pltpu.async_copy(
                    wt_hbm.at[:, pl.ds(col0_of(j + NBUF_R), CHUNK)],
                    col_v.at[grb], gsem.at[grb],
                )
            return carry

        lax.fori_loop(0, niter, body, 0)
        for b in range(NBUF):
            pltpu.make_async_copy(
                row_v.at[b], out_hbm.at[pl.ds(0, CHUNK)], ssem.at[b]
            ).wait()

    return transpose_kernel


def _make_gather():
    mesh = plsc.VectorSubcoreMesh(core_axis_name="c", subcore_axis_name="s")

    @functools.partial(
        pl.kernel,
        mesh=mesh,
        out_type=jax.ShapeDtypeStruct(
            (HIST * DIM // 8, BATCH // CHUNK, 8, CHUNK), jnp.float32
        ),
        scratch_types=[
            pltpu.VMEM((NCHUNK, CHUNK), jnp.int32),
            pltpu.VMEM((NBUF_G, CHUNK, CHUNK), jnp.float32),
            pltpu.VMEM((NBUF, 8, 8, CHUNK), jnp.float32),
            pltpu.SemaphoreType.DMA((NBUF_G,)),
            pltpu.SemaphoreType.DMA((NBUF,)),
        ],
        compiler_params=pltpu.CompilerParams(
            use_tc_tiling_on_sc=False, needs_layout_passes=False
        ),
    )
    def gather_kernel(idx_hbm, table_hbm, out_hbm,
                      idx_v, rows_v, outT_v, gsem, ssem):
        wid = lax.axis_index("s") * 2 + lax.axis_index("c")
        pltpu.sync_copy(idx_hbm.at[wid], idx_v)

        for b in range(NBUF_G):
            pltpu.async_copy(
                table_hbm.at[idx_v.at[b]], rows_v.at[b], gsem.at[b]
            )

        iota = _iota16()
        zero = iota * 0
        dlo = lax.rem(iota, 8)
        dhis = [lax.div(iota, 8) + dg * 2 for dg in range(4)]

        def body(c, carry):
            grb = lax.rem(c, NBUF_G)
            rb = lax.rem(c, NBUF)
            t = c // BC_PER_W
            bcg = wid * BC_PER_W + lax.rem(c, BC_PER_W)
            pltpu.make_async_copy(
                table_hbm.at[pl.ds(0, CHUNK)], rows_v.at[grb], gsem.at[grb]
            ).wait()
            @pl.when(c >= NBUF)
            def _():
                pltpu.make_async_copy(
                    outT_v.at[rb], out_hbm.at[pl.ds(0, 8), 0], ssem.at[rb]
                ).wait()
            src = rows_v.at[grb]
            dst = outT_v.at[rb]

            @plsc.parallel_loop(0, CHUNK, unroll=16)
            def _transpose_chunk(k):
                kvec = zero + k
                for dg in range(4):
                    v = src[k, pl.ds(dg * 16, 16)]
                    plsc.store_scatter(dst, [dhis[dg], dlo, kvec], v)
            pltpu.async_copy(
                outT_v.at[rb], out_hbm.at[pl.ds(t * 8, 8), bcg], ssem.at[rb]
            )
            @pl.when(c + NBUF_G < NCHUNK)
            def _():
                pltpu.async_copy(
                    table_hbm.at[idx_v.at[c + NBUF_G]], rows_v.at[grb],
                    gsem.at[grb],
                )
            return carry

        lax.fori_loop(0, NCHUNK, body, 0)
        for b in range(NBUF):
            pltpu.make_async_copy(
                outT_v.at[b], out_hbm.at[pl.ds(0, 8), 0], ssem.at[b]
            ).wait()

    return gather_kernel


_transpose = _make_transpose()
_gather = _make_gather()


def kernel(x, weight):
    table = _transpose(weight.T)
    xt = x.T.astype(jnp.int32)
    xw = xt.reshape(HIST, NW, BC_PER_W, CHUNK)
    xw = xw.transpose(1, 0, 2, 3).reshape(NW, NCHUNK, CHUNK)
    out = _gather(xw, table)
    v = out.reshape(HIST, 8, BATCH // CHUNK, 8, CHUNK)
    return v.transpose(2, 4, 0, 1, 3).reshape(BATCH, HIST, DIM)


# speedup vs baseline: 1.3141x; 1.3141x over previous
import functools

import jax
import jax.numpy as jnp
from jax import lax
from jax.experimental import pallas as pl
from jax.experimental.pallas import tpu as pltpu
from jax.experimental.pallas import tpu_sc as plsc

NUM_EMB = 1_000_000
DIM = 64
BATCH = 16384
HIST = 20
B_TOTAL = BATCH * HIST
NW = 32
B_PER_W = B_TOTAL // NW
CHUNK = 128
NCHUNK = B_PER_W // CHUNK
NBUF = 8
NGROUP = NCHUNK // NBUF


def _make_kernel():
    mesh = plsc.VectorSubcoreMesh(core_axis_name="c", subcore_axis_name="s")

    @functools.partial(
        pl.kernel,
        mesh=mesh,
        out_type=jax.ShapeDtypeStruct((B_TOTAL, DIM), jnp.float32),
        scratch_types=[
            pltpu.VMEM((NCHUNK, CHUNK), jnp.int32),
            pltpu.VMEM((NBUF, CHUNK, DIM), jnp.float32),
            pltpu.SemaphoreType.DMA((NBUF,)),
            pltpu.SemaphoreType.DMA((NBUF,)),
        ],
        compiler_params=pltpu.CompilerParams(use_tc_tiling_on_sc=False),
    )
    def gather_kernel(idx_hbm, table_hbm, out_hbm, idx_v, rows_v, gsem, ssem):
        wid = lax.axis_index("s") * 2 + lax.axis_index("c")
        base = wid * B_PER_W
        pltpu.sync_copy(idx_hbm.at[wid], idx_v)

        def group(g, carry):
            j0 = g * NBUF
            descs = []
            for b in range(NBUF):
                @pl.when(g > 0)
                def _wait_store(b=b):
                    pltpu.make_async_copy(
                        rows_v.at[b],
                        out_hbm.at[pl.ds(base, CHUNK)],
                        ssem.at[b],
                    ).wait()
                descs.append(
                    pltpu.async_copy(
                        table_hbm.at[idx_v.at[j0 + b]], rows_v.at[b], gsem.at[b]
                    )
                )
            for b in range(NBUF):
                descs[b].wait()
                pltpu.async_copy(
                    rows_v.at[b],
                    out_hbm.at[pl.ds(base + (j0 + b) * CHUNK, CHUNK)],
                    ssem.at[b],
                )
            return carry

        lax.fori_loop(0, NGROUP, group, 0)
        for b in range(NBUF):
            pltpu.make_async_copy(
                rows_v.at[b],
                out_hbm.at[pl.ds(base, CHUNK)],
                ssem.at[b],
            ).wait()

    return gather_kernel


_gather = _make_kernel()


def kernel(x, weight):
    idx = x.reshape(NW, NCHUNK, CHUNK).astype(jnp.int32)
    out = _gather(idx, weight)
    return out.reshape(BATCH, HIST, DIM)
